# baseline (speedup 1.0000x reference)
import functools

import jax
import jax.numpy as jnp
from jax import lax
from jax.experimental import pallas as pl
from jax.experimental.pallas import tpu as pltpu
from jax.experimental.pallas import tpu_sc as plsc

N = 10000
E = 160000
NP = 10112
NTILES = 32
BATCH = 256
NB = 20
EPAD = NTILES * NB * BATCH
STRIPE = NP // 16
ZR = 128


def _leaky(x):
    return jnp.where(x >= 0, x, 0.01 * x)


@functools.cache
def _make_prop(d, with_gather=True):
    mesh = plsc.VectorSubcoreMesh(core_axis_name="c", subcore_axis_name="s",
                                  num_cores=2, num_subcores=16)
    scratch = [
        pltpu.VMEM((NB, BATCH), jnp.int32),
        pltpu.VMEM((BATCH, d), jnp.float32),
        pltpu.VMEM((BATCH, d), jnp.float32),
        pltpu.VMEM((ZR, d), jnp.float32),
        pltpu.VMEM_SHARED((NP, d), jnp.float32),
        pltpu.SemaphoreType.DMA,
        pltpu.SemaphoreType.DMA,
        pltpu.SemaphoreType.DMA,
        pltpu.SemaphoreType.DMA,
    ]
    if with_gather:
        scratch.insert(0, pltpu.VMEM((NB, BATCH), jnp.int32))

    def body(*refs):
        if with_gather:
            (table, fill, src_h, dst_h, out, src_v, dst_v, rows0, rows1,
             zbuf, acc, gs0, gs1, ss0, ss1) = refs
        else:
            (fill, dst_h, out, dst_v, rows0, rows1,
             zbuf, acc, gs0, gs1, ss0, ss1) = refs
        rows = (rows0, rows1)
        gsem = (gs0, gs1)
        ssem = (ss0, ss1)
        c = lax.axis_index("c")
        s = lax.axis_index("s")
        wid = c * 16 + s
        row0 = s * STRIPE

        pltpu.sync_copy(fill.at[0, pl.ds(0, ZR)], zbuf)
        if with_gather:
            pltpu.sync_copy(src_h.at[wid], src_v)
        else:
            pltpu.sync_copy(fill.at[1], rows0)
        pltpu.sync_copy(dst_h.at[wid], dst_v)

        off = 0
        while off < STRIPE:
            ln = min(ZR, STRIPE - off)
            pltpu.sync_copy(zbuf.at[pl.ds(0, ln)],
                            acc.at[pl.ds(row0 + off, ln)])
            off += ln
        plsc.subcore_barrier()

        if with_gather:
            gd = {}
            sd = {}
            gd[0] = pltpu.async_copy(table.at[src_v.at[0]], rows[0], gsem[0])
            for j in range(NB):
                cur = j & 1
                nxt = 1 - cur
                if j + 1 < NB:
                    if j - 1 >= 0:
                        sd[j - 1].wait()
                    gd[j + 1] = pltpu.async_copy(
                        table.at[src_v.at[j + 1]], rows[nxt], gsem[nxt])
                gd[j].wait()
                sd[j] = pltpu.async_copy(rows[cur], acc.at[dst_v.at[j]],
                                         ssem[cur], add=True)
            if NB >= 2:
                sd[NB - 2].wait()
            sd[NB - 1].wait()
        else:
            sd = {}
            for j in range(NB):
                sd[j] = pltpu.async_copy(rows0, acc.at[dst_v.at[j]],
                                         ssem[0], add=True)
            for j in range(NB):
                sd[j].wait()
        plsc.subcore_barrier()

        off = 0
        while off < STRIPE:
            ln = min(ZR, STRIPE - off)
            pltpu.sync_copy(acc.at[pl.ds(row0 + off, ln)],
                            zbuf.at[pl.ds(0, ln)])
            pltpu.sync_copy(zbuf.at[pl.ds(0, ln)],
                            out.at[c, pl.ds(row0 + off, ln)])
            off += ln

    return pl.kernel(
        body,
        out_type=jax.ShapeDtypeStruct((2, NP, d), jnp.float32),
        mesh=mesh,
        scratch_types=scratch,
        compiler_params=pltpu.CompilerParams(use_tc_tiling_on_sc=False),
    )


def _prop(d, table, fill, edges):
    src, dst = edges
    return _make_prop(d)(table, fill, src, dst)


def _prop_deg(fill, edges):
    return _make_prop(16, with_gather=False)(fill, edges[1])


def _prep_body(degp_ref, pos_ref, dinv_ref, xs0_ref):
    deg = degp_ref[0, :N, :] + degp_ref[1, :N, :] + 1.0
    di = lax.rsqrt(deg)
    dinv_ref[...] = di[:, 0:1]
    x16 = jnp.concatenate(
        [pos_ref[...], jnp.zeros((N, 13), jnp.float32)], axis=1)
    xs0_ref[...] = x16 * di


def _make_conv_body(nin_parts, nout_parts):
    def body(*refs):
        k = 0
        parts = []
        for _ in range(nin_parts):
            p_ref = refs[k]
            xs_ref = refs[k + 1]
            parts.append((p_ref, xs_ref))
            k += 2
        dinv_ref, w_ref, g_ref, bt_ref = refs[k:k + 4]
        outs = refs[k + 4:]
        di = dinv_ref[...]
        y = None
        c0 = 0
        for p_ref, xs_ref in parts:
            dpart = xs_ref.shape[1]
            u = di * (p_ref[0, :N, :] + p_ref[1, :N, :] + xs_ref[...])
            contrib = jnp.dot(u, w_ref[c0:c0 + dpart, :],
                              preferred_element_type=jnp.float32)
            y = contrib if y is None else y + contrib
            c0 += dpart
        m = jnp.mean(y, axis=0, keepdims=True)
        v = jnp.mean((y - m) ** 2, axis=0, keepdims=True)
        yn = (y - m) * lax.rsqrt(v + 1e-5) * g_ref[...] + bt_ref[...]
        x_next = di * _leaky(yn)
        do = y.shape[1]
        w = do // nout_parts
        for i, o_ref in enumerate(outs):
            o_ref[...] = x_next[:, i * w:(i + 1) * w]
    return body


def _conv4_body(pa_ref, xsa_ref, pb_ref, xsb_ref, dinv_ref, w_ref, g_ref,
                bt_ref, q_ref):
    di = dinv_ref[...]
    ua = di * (pa_ref[0, :N, :] + pa_ref[1, :N, :] + xsa_ref[...])
    ub = di * (pb_ref[0, :N, :] + pb_ref[1, :N, :] + xsb_ref[...])
    y = (jnp.dot(ua, w_ref[0:64, :], preferred_element_type=jnp.float32)
         + jnp.dot(ub, w_ref[64:128, :], preferred_element_type=jnp.float32))
    m = jnp.mean(y, axis=0, keepdims=True)
    v = jnp.mean((y - m) ** 2, axis=0, keepdims=True)
    a = g_ref[...] * lax.rsqrt(v + 1e-5)
    hi = jnp.max(y, axis=0, keepdims=True)
    lo = jnp.min(y, axis=0, keepdims=True)
    pooled = jnp.where(a >= 0, hi, lo)
    q_ref[...] = _leaky((pooled - m) * a + bt_ref[...])


def _mlp_body(q_ref, w1_ref, b1_ref, g5_ref, bt5_ref,
              w2_ref, b2_ref, g6_ref, bt6_ref, out_ref):
    h = jnp.dot(q_ref[...], w1_ref[...], preferred_element_type=jnp.float32)
    h = _leaky((h + b1_ref[...]) * g5_ref[...] + bt5_ref[...])
    o = jnp.dot(h, w2_ref[...], preferred_element_type=jnp.float32)
    out_ref[...] = (o + b2_ref[...]) * g6_ref[...] + bt6_ref[...]


_PREP = pl.pallas_call(
    _prep_body,
    out_shape=(jax.ShapeDtypeStruct((N, 1), jnp.float32),
               jax.ShapeDtypeStruct((N, 16), jnp.float32)),
)


def _make_conv(nin_parts, nout_parts, do):
    w = do // nout_parts
    return pl.pallas_call(
        _make_conv_body(nin_parts, nout_parts),
        out_shape=tuple(jax.ShapeDtypeStruct((N, w), jnp.float32)
                        for _ in range(nout_parts)))


_FT = 256
_CONV4 = pl.pallas_call(
    _conv4_body,
    grid=(2048 // _FT,),
    in_specs=[
        pl.BlockSpec((2, NP, 64), lambda i: (0, 0, 0)),
        pl.BlockSpec((N, 64), lambda i: (0, 0)),
        pl.BlockSpec((2, NP, 64), lambda i: (0, 0, 0)),
        pl.BlockSpec((N, 64), lambda i: (0, 0)),
        pl.BlockSpec((N, 1), lambda i: (0, 0)),
        pl.BlockSpec((128, _FT), lambda i: (0, i)),
        pl.BlockSpec((1, _FT), lambda i: (0, i)),
        pl.BlockSpec((1, _FT), lambda i: (0, i)),
    ],
    out_specs=pl.BlockSpec((1, _FT), lambda i: (0, i)),
    out_shape=jax.ShapeDtypeStruct((1, 2048), jnp.float32),
)

_MLP = pl.pallas_call(
    _mlp_body, out_shape=jax.ShapeDtypeStruct((1, 4096), jnp.float32))


def kernel(pos, edge_index, W0, b0, g0, bt0, W1, b1, g1, bt1, W2, b2, g2, bt2,
           W3, b3, g3, bt3, W4, b4, g4, bt4, L1w, L1b, g5, bt5,
           L2w, L2b, g6, bt6):
    edges = (
        jnp.concatenate([edge_index[0],
                         jnp.zeros((EPAD - E,), jnp.int32)]).reshape(
                             NTILES, NB, BATCH),
        jnp.concatenate([edge_index[1],
                         jnp.full((EPAD - E,), N, jnp.int32)]).reshape(
                             NTILES, NB, BATCH))
    fill16 = jnp.zeros((1, ZR, 16), jnp.float32)
    fill64 = jnp.zeros((1, ZR, 64), jnp.float32)
    fill_deg = jnp.concatenate(
        [jnp.zeros((1, BATCH, 16), jnp.float32),
         jnp.ones((1, BATCH, 16), jnp.float32)], axis=0)

    degp = _prop_deg(fill_deg, edges)
    dinv, xs = _PREP(degp, pos)

    W0p = jnp.pad(W0, ((0, 13), (0, 0)))

    p = _prop(16, xs, fill16, edges)
    (xs,) = _make_conv(1, 1, 64)(p, xs, dinv, W0p,
                                 g0.reshape(1, 64), bt0.reshape(1, 64))
    for W, g, bt in ((W1, g1, bt1), (W2, g2, bt2)):
        p = _prop(64, xs, fill64, edges)
        (xs,) = _make_conv(1, 1, 64)(p, xs, dinv, W,
                                     g.reshape(1, 64), bt.reshape(1, 64))
    p = _prop(64, xs, fill64, edges)
    xsa, xsb = _make_conv(1, 2, 128)(p, xs, dinv, W3,
                                     g3.reshape(1, 128), bt3.reshape(1, 128))
    pa = _prop(64, xsa, fill64, edges)
    pb = _prop(64, xsb, fill64, edges)
    q = _CONV4(pa, xsa, pb, xsb, dinv, W4,
               g4.reshape(1, 2048), bt4.reshape(1, 2048))
    out = _MLP(q, L1w, L1b.reshape(1, 512), g5.reshape(1, 512),
               bt5.reshape(1, 512), L2w, L2b.reshape(1, 4096),
               g6.reshape(1, 4096), bt6.reshape(1, 4096))
    return out.reshape(4096)


# speedup vs baseline: 1.0566x; 1.0566x over previous
import functools

import jax
import jax.numpy as jnp
from jax import lax
from jax.experimental import pallas as pl
from jax.experimental.pallas import tpu as pltpu
from jax.experimental.pallas import tpu_sc as plsc

N = 10000
E = 160000
NP = 10112
NTILES = 32
BATCH = 512
NB = 10
EPAD = NTILES * NB * BATCH
STRIPE = NP // 16
ZR = 128


def _leaky(x):
    return jnp.where(x >= 0, x, 0.01 * x)


@functools.cache
def _make_prop(d, with_gather=True):
    mesh = plsc.VectorSubcoreMesh(core_axis_name="c", subcore_axis_name="s",
                                  num_cores=2, num_subcores=16)
    scratch = [
        pltpu.VMEM((NB, BATCH), jnp.int32),
        pltpu.VMEM((BATCH, d), jnp.float32),
        pltpu.VMEM((BATCH, d), jnp.float32),
        pltpu.VMEM((ZR, d), jnp.float32),
        pltpu.VMEM_SHARED((NP, d), jnp.float32),
        pltpu.SemaphoreType.DMA,
        pltpu.SemaphoreType.DMA,
        pltpu.SemaphoreType.DMA,
        pltpu.SemaphoreType.DMA,
    ]
    if with_gather:
        scratch.insert(0, pltpu.VMEM((NB, BATCH), jnp.int32))

    def body(*refs):
        if with_gather:
            (table, fill, src_h, dst_h, out, src_v, dst_v, rows0, rows1,
             zbuf, acc, gs0, gs1, ss0, ss1) = refs
        else:
            (fill, dst_h, out, dst_v, rows0, rows1,
             zbuf, acc, gs0, gs1, ss0, ss1) = refs
        rows = (rows0, rows1)
        gsem = (gs0, gs1)
        ssem = (ss0, ss1)
        c = lax.axis_index("c")
        s = lax.axis_index("s")
        wid = c * 16 + s
        row0 = s * STRIPE

        pltpu.sync_copy(fill.at[0, pl.ds(0, ZR)], zbuf)
        if with_gather:
            pltpu.sync_copy(src_h.at[wid], src_v)
        else:
            pltpu.sync_copy(fill.at[1], rows0)
        pltpu.sync_copy(dst_h.at[wid], dst_v)

        off = 0
        while off < STRIPE:
            ln = min(ZR, STRIPE - off)
            pltpu.sync_copy(zbuf.at[pl.ds(0, ln)],
                            acc.at[pl.ds(row0 + off, ln)])
            off += ln
        plsc.subcore_barrier()

        if with_gather:
            gd = {}
            sd = {}
            gd[0] = pltpu.async_copy(table.at[src_v.at[0]], rows[0], gsem[0])
            for j in range(NB):
                cur = j & 1
                nxt = 1 - cur
                if j + 1 < NB:
                    if j - 1 >= 0:
                        sd[j - 1].wait()
                    gd[j + 1] = pltpu.async_copy(
                        table.at[src_v.at[j + 1]], rows[nxt], gsem[nxt])
                gd[j].wait()
                sd[j] = pltpu.async_copy(rows[cur], acc.at[dst_v.at[j]],
                                         ssem[cur], add=True)
            if NB >= 2:
                sd[NB - 2].wait()
            sd[NB - 1].wait()
        else:
            sd = {}
            for j in range(NB):
                sd[j] = pltpu.async_copy(rows0, acc.at[dst_v.at[j]],
                                         ssem[0], add=True)
            for j in range(NB):
                sd[j].wait()
        plsc.subcore_barrier()

        pltpu.sync_copy(acc.at[pl.ds(row0, STRIPE)],
                        out.at[c, pl.ds(row0, STRIPE)])

    return pl.kernel(
        body,
        out_type=jax.ShapeDtypeStruct((2, NP, d), jnp.float32),
        mesh=mesh,
        scratch_types=scratch,
        compiler_params=pltpu.CompilerParams(use_tc_tiling_on_sc=False),
    )


def _prop(d, table, fill, edges):
    src, dst = edges
    return _make_prop(d)(table, fill, src, dst)


def _prop_deg(fill, edges):
    return _make_prop(16, with_gather=False)(fill, edges[1])


def _prep_body(degp_ref, pos_ref, dinv_ref, xs0_ref):
    deg = degp_ref[0, :N, :] + degp_ref[1, :N, :] + 1.0
    di = lax.rsqrt(deg)
    dinv_ref[...] = di[:, 0:1]
    x16 = jnp.concatenate(
        [pos_ref[...], jnp.zeros((N, 13), jnp.float32)], axis=1)
    xs0_ref[...] = x16 * di


def _make_conv_body(nin_parts, nout_parts):
    def body(*refs):
        k = 0
        parts = []
        for _ in range(nin_parts):
            p_ref = refs[k]
            xs_ref = refs[k + 1]
            parts.append((p_ref, xs_ref))
            k += 2
        dinv_ref, w_ref, g_ref, bt_ref = refs[k:k + 4]
        outs = refs[k + 4:]
        di = dinv_ref[...]
        y = None
        c0 = 0
        for p_ref, xs_ref in parts:
            dpart = xs_ref.shape[1]
            u = di * (p_ref[0, :N, :] + p_ref[1, :N, :] + xs_ref[...])
            contrib = jnp.dot(u, w_ref[c0:c0 + dpart, :],
                              preferred_element_type=jnp.float32)
            y = contrib if y is None else y + contrib
            c0 += dpart
        m = jnp.mean(y, axis=0, keepdims=True)
        v = jnp.mean((y - m) ** 2, axis=0, keepdims=True)
        yn = (y - m) * lax.rsqrt(v + 1e-5) * g_ref[...] + bt_ref[...]
        x_next = di * _leaky(yn)
        do = y.shape[1]
        w = do // nout_parts
        for i, o_ref in enumerate(outs):
            o_ref[...] = x_next[:, i * w:(i + 1) * w]
    return body


def _conv4_body(pa_ref, xsa_ref, pb_ref, xsb_ref, dinv_ref, w_ref, g_ref,
                bt_ref, q_ref):
    di = dinv_ref[...]
    ua = di * (pa_ref[0, :N, :] + pa_ref[1, :N, :] + xsa_ref[...])
    ub = di * (pb_ref[0, :N, :] + pb_ref[1, :N, :] + xsb_ref[...])
    y = (jnp.dot(ua, w_ref[0:64, :], preferred_element_type=jnp.float32)
         + jnp.dot(ub, w_ref[64:128, :], preferred_element_type=jnp.float32))
    m = jnp.mean(y, axis=0, keepdims=True)
    v = jnp.mean((y - m) ** 2, axis=0, keepdims=True)
    a = g_ref[...] * lax.rsqrt(v + 1e-5)
    hi = jnp.max(y, axis=0, keepdims=True)
    lo = jnp.min(y, axis=0, keepdims=True)
    pooled = jnp.where(a >= 0, hi, lo)
    q_ref[...] = _leaky((pooled - m) * a + bt_ref[...])


def _mlp_body(q_ref, w1_ref, b1_ref, g5_ref, bt5_ref,
              w2_ref, b2_ref, g6_ref, bt6_ref, out_ref):
    h = jnp.dot(q_ref[...], w1_ref[...], preferred_element_type=jnp.float32)
    h = _leaky((h + b1_ref[...]) * g5_ref[...] + bt5_ref[...])
    o = jnp.dot(h, w2_ref[...], preferred_element_type=jnp.float32)
    out_ref[...] = (o + b2_ref[...]) * g6_ref[...] + bt6_ref[...]


_PREP = pl.pallas_call(
    _prep_body,
    out_shape=(jax.ShapeDtypeStruct((N, 1), jnp.float32),
               jax.ShapeDtypeStruct((N, 16), jnp.float32)),
)


def _make_conv(nin_parts, nout_parts, do):
    w = do // nout_parts
    return pl.pallas_call(
        _make_conv_body(nin_parts, nout_parts),
        out_shape=tuple(jax.ShapeDtypeStruct((N, w), jnp.float32)
                        for _ in range(nout_parts)))


_FT = 256
_CONV4 = pl.pallas_call(
    _conv4_body,
    grid=(2048 // _FT,),
    in_specs=[
        pl.BlockSpec((2, NP, 64), lambda i: (0, 0, 0)),
        pl.BlockSpec((N, 64), lambda i: (0, 0)),
        pl.BlockSpec((2, NP, 64), lambda i: (0, 0, 0)),
        pl.BlockSpec((N, 64), lambda i: (0, 0)),
        pl.BlockSpec((N, 1), lambda i: (0, 0)),
        pl.BlockSpec((128, _FT), lambda i: (0, i)),
        pl.BlockSpec((1, _FT), lambda i: (0, i)),
        pl.BlockSpec((1, _FT), lambda i: (0, i)),
    ],
    out_specs=pl.BlockSpec((1, _FT), lambda i: (0, i)),
    out_shape=jax.ShapeDtypeStruct((1, 2048), jnp.float32),
)

_MLP = pl.pallas_call(
    _mlp_body, out_shape=jax.ShapeDtypeStruct((1, 4096), jnp.float32))


def kernel(pos, edge_index, W0, b0, g0, bt0, W1, b1, g1, bt1, W2, b2, g2, bt2,
           W3, b3, g3, bt3, W4, b4, g4, bt4, L1w, L1b, g5, bt5,
           L2w, L2b, g6, bt6):
    edges = (
        jnp.concatenate([edge_index[0],
                         jnp.zeros((EPAD - E,), jnp.int32)]).reshape(
                             NTILES, NB, BATCH),
        jnp.concatenate([edge_index[1],
                         jnp.full((EPAD - E,), N, jnp.int32)]).reshape(
                             NTILES, NB, BATCH))
    fill16 = jnp.zeros((1, ZR, 16), jnp.float32)
    fill64 = jnp.zeros((1, ZR, 64), jnp.float32)
    fill_deg = jnp.concatenate(
        [jnp.zeros((1, BATCH, 16), jnp.float32),
         jnp.ones((1, BATCH, 16), jnp.float32)], axis=0)

    degp = _prop_deg(fill_deg, edges)
    dinv, xs = _PREP(degp, pos)

    W0p = jnp.pad(W0, ((0, 13), (0, 0)))

    p = _prop(16, xs, fill16, edges)
    (xs,) = _make_conv(1, 1, 64)(p, xs, dinv, W0p,
                                 g0.reshape(1, 64), bt0.reshape(1, 64))
    for W, g, bt in ((W1, g1, bt1), (W2, g2, bt2)):
        p = _prop(64, xs, fill64, edges)
        (xs,) = _make_conv(1, 1, 64)(p, xs, dinv, W,
                                     g.reshape(1, 64), bt.reshape(1, 64))
    p = _prop(64, xs, fill64, edges)
    xsa, xsb = _make_conv(1, 2, 128)(p, xs, dinv, W3,
                                     g3.reshape(1, 128), bt3.reshape(1, 128))
    pa = _prop(64, xsa, fill64, edges)
    pb = _prop(64, xsb, fill64, edges)
    q = _CONV4(pa, xsa, pb, xsb, dinv, W4,
               g4.reshape(1, 2048), bt4.reshape(1, 2048))
    out = _MLP(q, L1w, L1b.reshape(1, 512), g5.reshape(1, 512),
               bt5.reshape(1, 512), L2w, L2b.reshape(1, 4096),
               g6.reshape(1, 4096), bt6.reshape(1, 4096))
    return out.reshape(4096)


# speedup vs baseline: 1.0612x; 1.0043x over previous
import functools

import jax
import jax.numpy as jnp
from jax import lax
from jax.experimental import pallas as pl
from jax.experimental.pallas import tpu as pltpu
from jax.experimental.pallas import tpu_sc as plsc

N = 10000
E = 160000
NP = 10112
NTILES = 32
BATCH = 512
NB = 10
EPAD = NTILES * NB * BATCH
STRIPE = NP // 16
ZR = 128


def _leaky(x):
    return jnp.where(x >= 0, x, 0.01 * x)


@functools.cache
def _make_prop(d, with_gather=True):
    mesh = plsc.VectorSubcoreMesh(core_axis_name="c", subcore_axis_name="s",
                                  num_cores=2, num_subcores=16)
    scratch = [
        pltpu.VMEM((NB, BATCH), jnp.int32),
        pltpu.VMEM((BATCH, d), jnp.float32),
        pltpu.VMEM((BATCH, d), jnp.float32),
        pltpu.VMEM((ZR, d), jnp.float32),
        pltpu.VMEM_SHARED((NP, d), jnp.float32),
        pltpu.SemaphoreType.DMA,
        pltpu.SemaphoreType.DMA,
        pltpu.SemaphoreType.DMA,
        pltpu.SemaphoreType.DMA,
    ]
    if with_gather:
        scratch.insert(0, pltpu.VMEM((NB, BATCH), jnp.int32))

    def body(*refs):
        if with_gather:
            (table, fill, src_h, dst_h, out, src_v, dst_v, rows0, rows1,
             zbuf, acc, gs0, gs1, ss0, ss1) = refs
        else:
            (fill, dst_h, out, dst_v, rows0, rows1,
             zbuf, acc, gs0, gs1, ss0, ss1) = refs
        rows = (rows0, rows1)
        gsem = (gs0, gs1)
        ssem = (ss0, ss1)
        c = lax.axis_index("c")
        s = lax.axis_index("s")
        wid = c * 16 + s
        row0 = s * STRIPE

        pltpu.sync_copy(fill.at[0, pl.ds(0, ZR)], zbuf)
        if with_gather:
            pltpu.sync_copy(src_h.at[wid], src_v)
        else:
            pltpu.sync_copy(fill.at[1], rows0)
        pltpu.sync_copy(dst_h.at[wid], dst_v)

        off = 0
        while off < STRIPE:
            ln = min(ZR, STRIPE - off)
            pltpu.sync_copy(zbuf.at[pl.ds(0, ln)],
                            acc.at[pl.ds(row0 + off, ln)])
            off += ln
        plsc.subcore_barrier()

        if with_gather:
            gd = {}
            sd = {}
            gd[0] = pltpu.async_copy(table.at[src_v.at[0]], rows[0], gsem[0])
            for j in range(NB):
                cur = j & 1
                nxt = 1 - cur
                if j + 1 < NB:
                    if j - 1 >= 0:
                        sd[j - 1].wait()
                    gd[j + 1] = pltpu.async_copy(
                        table.at[src_v.at[j + 1]], rows[nxt], gsem[nxt])
                gd[j].wait()
                sd[j] = pltpu.async_copy(rows[cur], acc.at[dst_v.at[j]],
                                         ssem[cur], add=True)
            if NB >= 2:
                sd[NB - 2].wait()
            sd[NB - 1].wait()
        else:
            sd = {}
            for j in range(NB):
                sd[j] = pltpu.async_copy(rows0, acc.at[dst_v.at[j]],
                                         ssem[0], add=True)
            for j in range(NB):
                sd[j].wait()
        plsc.subcore_barrier()

        pltpu.sync_copy(acc.at[pl.ds(row0, STRIPE)],
                        out.at[c, pl.ds(row0, STRIPE)])

    return pl.kernel(
        body,
        out_type=jax.ShapeDtypeStruct((2, NP, d), jnp.float32),
        mesh=mesh,
        scratch_types=scratch,
        compiler_params=pltpu.CompilerParams(use_tc_tiling_on_sc=False),
    )


def _prop(d, table, fill, edges):
    src, dst = edges
    return _make_prop(d)(table, fill, src, dst)


def _prop_deg(fill, edges):
    return _make_prop(16, with_gather=False)(fill, edges[1])


def _prep_body(degp_ref, pos_ref, dinv_ref, xs0_ref):
    deg = degp_ref[0, :N, :] + degp_ref[1, :N, :] + 1.0
    di = lax.rsqrt(deg)
    dinv_ref[...] = di[:, 0:1]
    x16 = jnp.concatenate(
        [pos_ref[...], jnp.zeros((N, 13), jnp.float32)], axis=1)
    xs0_ref[...] = x16 * di


def _make_conv_body(nin_parts, nout_parts):
    def body(*refs):
        k = 0
        parts = []
        for _ in range(nin_parts):
            p_ref = refs[k]
            xs_ref = refs[k + 1]
            parts.append((p_ref, xs_ref))
            k += 2
        dinv_ref, w_ref, g_ref, bt_ref = refs[k:k + 4]
        outs = refs[k + 4:]
        di = dinv_ref[...]
        y = None
        c0 = 0
        for p_ref, xs_ref in parts:
            dpart = xs_ref.shape[1]
            u = di * (p_ref[0, :N, :] + p_ref[1, :N, :] + xs_ref[...])
            contrib = jnp.dot(u, w_ref[c0:c0 + dpart, :],
                              preferred_element_type=jnp.float32)
            y = contrib if y is None else y + contrib
            c0 += dpart
        m = jnp.mean(y, axis=0, keepdims=True)
        v = jnp.mean((y - m) ** 2, axis=0, keepdims=True)
        yn = (y - m) * lax.rsqrt(v + 1e-5) * g_ref[...] + bt_ref[...]
        x_next = di * _leaky(yn)
        do = y.shape[1]
        w = do // nout_parts
        for i, o_ref in enumerate(outs):
            o_ref[...] = x_next[:, i * w:(i + 1) * w]
    return body


def _conv4_body(pa_ref, xsa_ref, pb_ref, xsb_ref, dinv_ref, w_ref, g_ref,
                bt_ref, q_ref):
    di = dinv_ref[...]
    ua = di * (pa_ref[0, :N, :] + pa_ref[1, :N, :] + xsa_ref[...])
    ub = di * (pb_ref[0, :N, :] + pb_ref[1, :N, :] + xsb_ref[...])
    y = (jnp.dot(ua, w_ref[0:64, :], preferred_element_type=jnp.float32)
         + jnp.dot(ub, w_ref[64:128, :], preferred_element_type=jnp.float32))
    m = jnp.mean(y, axis=0, keepdims=True)
    v = jnp.mean((y - m) ** 2, axis=0, keepdims=True)
    a = g_ref[...] * lax.rsqrt(v + 1e-5)
    hi = jnp.max(y, axis=0, keepdims=True)
    lo = jnp.min(y, axis=0, keepdims=True)
    pooled = jnp.where(a >= 0, hi, lo)
    q_ref[...] = _leaky((pooled - m) * a + bt_ref[...])


def _mlp_body(q_ref, w1_ref, b1_ref, g5_ref, bt5_ref,
              w2_ref, b2_ref, g6_ref, bt6_ref, out_ref):
    h = jnp.dot(q_ref[...], w1_ref[...], preferred_element_type=jnp.float32)
    h = _leaky((h + b1_ref[...]) * g5_ref[...] + bt5_ref[...])
    o = jnp.dot(h, w2_ref[...], preferred_element_type=jnp.float32)
    out_ref[...] = (o + b2_ref[...]) * g6_ref[...] + bt6_ref[...]


_PREP = pl.pallas_call(
    _prep_body,
    out_shape=(jax.ShapeDtypeStruct((N, 1), jnp.float32),
               jax.ShapeDtypeStruct((N, 16), jnp.float32)),
)


def _make_conv(nin_parts, nout_parts, do):
    w = do // nout_parts
    return pl.pallas_call(
        _make_conv_body(nin_parts, nout_parts),
        out_shape=tuple(jax.ShapeDtypeStruct((N, w), jnp.float32)
                        for _ in range(nout_parts)))


_FT = 256
_CONV4 = pl.pallas_call(
    _conv4_body,
    grid=(2048 // _FT,),
    in_specs=[
        pl.BlockSpec((2, NP, 64), lambda i: (0, 0, 0)),
        pl.BlockSpec((N, 64), lambda i: (0, 0)),
        pl.BlockSpec((2, NP, 64), lambda i: (0, 0, 0)),
        pl.BlockSpec((N, 64), lambda i: (0, 0)),
        pl.BlockSpec((N, 1), lambda i: (0, 0)),
        pl.BlockSpec((128, _FT), lambda i: (0, i)),
        pl.BlockSpec((1, _FT), lambda i: (0, i)),
        pl.BlockSpec((1, _FT), lambda i: (0, i)),
    ],
    out_specs=pl.BlockSpec((1, _FT), lambda i: (0, i)),
    out_shape=jax.ShapeDtypeStruct((1, 2048), jnp.float32),
)

_MLP = pl.pallas_call(
    _mlp_body, out_shape=jax.ShapeDtypeStruct((1, 4096), jnp.float32))


def kernel(pos, edge_index, W0, b0, g0, bt0, W1, b1, g1, bt1, W2, b2, g2, bt2,
           W3, b3, g3, bt3, W4, b4, g4, bt4, L1w, L1b, g5, bt5,
           L2w, L2b, g6, bt6):
    pad_dst = N + jnp.arange(EPAD - E, dtype=jnp.int32) % (NP - N)
    edges = (
        jnp.concatenate([edge_index[0],
                         jnp.zeros((EPAD - E,), jnp.int32)]).reshape(
                             NTILES, NB, BATCH),
        jnp.concatenate([edge_index[1], pad_dst]).reshape(
            NTILES, NB, BATCH))
    fill16 = jnp.zeros((1, ZR, 16), jnp.float32)
    fill64 = jnp.zeros((1, ZR, 64), jnp.float32)
    fill_deg = jnp.concatenate(
        [jnp.zeros((1, BATCH, 16), jnp.float32),
         jnp.ones((1, BATCH, 16), jnp.float32)], axis=0)

    degp = _prop_deg(fill_deg, edges)
    dinv, xs = _PREP(degp, pos)

    W0p = jnp.pad(W0, ((0, 13), (0, 0)))

    p = _prop(16, xs, fill16, edges)
    (xs,) = _make_conv(1, 1, 64)(p, xs, dinv, W0p,
                                 g0.reshape(1, 64), bt0.reshape(1, 64))
    for W, g, bt in ((W1, g1, bt1), (W2, g2, bt2)):
        p = _prop(64, xs, fill64, edges)
        (xs,) = _make_conv(1, 1, 64)(p, xs, dinv, W,
                                     g.reshape(1, 64), bt.reshape(1, 64))
    p = _prop(64, xs, fill64, edges)
    xsa, xsb = _make_conv(1, 2, 128)(p, xs, dinv, W3,
                                     g3.reshape(1, 128), bt3.reshape(1, 128))
    pa = _prop(64, xsa, fill64, edges)
    pb = _prop(64, xsb, fill64, edges)
    q = _CONV4(pa, xsa, pb, xsb, dinv, W4,
               g4.reshape(1, 2048), bt4.reshape(1, 2048))
    out = _MLP(q, L1w, L1b.reshape(1, 512), g5.reshape(1, 512),
               bt5.reshape(1, 512), L2w, L2b.reshape(1, 4096),
               g6.reshape(1, 4096), bt6.reshape(1, 4096))
    return out.reshape(4096)
